# baseline (speedup 1.0000x reference)
import functools

import jax
import jax.numpy as jnp
from jax import lax
from jax.experimental import pallas as pl
from jax.experimental.pallas import tpu as pltpu
from jax.experimental.pallas import tpu_sc as plsc

_N = 24000
_NCROP = 3072
_ROWS = 32
_NBINS = 2048
_MIN32 = -2147483648
_FLIP = 0x7FFFFFFF


def _sortable_key(dv):
    kv = lax.bitcast_convert_type(dv, jnp.int32)
    return jnp.where(kv < 0, kv ^ _FLIP, kv)


def _thresh_body(d_ref, t_ref, need_ref):
    kb = _sortable_key(d_ref[...]) ^ _MIN32
    nr = kb.shape[0]

    def step(i, carry):
        tval, r, cl, dm = carry
        bit = 31 - i
        bmask = jnp.int32(1) << bit
        matched = (kb & dm) == tval
        bitzero = (kb & bmask) == 0
        c0 = jnp.sum((matched & bitzero).astype(jnp.int32), axis=1,
                     keepdims=True)
        choose0 = r < c0
        tval = jnp.where(choose0, tval, tval | bmask)
        r = jnp.where(choose0, r, r - c0)
        cl = jnp.where(choose0, cl, cl + c0)
        return tval, r, cl, dm | bmask

    z = jnp.zeros((nr, 1), jnp.int32)
    tval, _, cl, _ = lax.fori_loop(
        0, 32, step, (z, z + jnp.int32(_NCROP - 1), z, jnp.int32(0)))
    t_s = tval ^ _MIN32
    t_ref[...] = jnp.broadcast_to(t_s, (nr, 16))
    need_ref[...] = jnp.broadcast_to(jnp.int32(_NCROP) - cl, (nr, 16))


def _thresholds(d2):
    half = _ROWS // 2
    return pl.pallas_call(
        _thresh_body,
        grid=(2,),
        in_specs=[pl.BlockSpec((half, _N), lambda i: (i, 0))],
        out_specs=(pl.BlockSpec((half, 16), lambda i: (i, 0)),
                   pl.BlockSpec((half, 16), lambda i: (i, 0))),
        out_shape=(jax.ShapeDtypeStruct((_ROWS, 16), jnp.int32),
                   jax.ShapeDtypeStruct((_ROWS, 16), jnp.int32)),
        compiler_params=pltpu.CompilerParams(
            dimension_semantics=("parallel",)),
    )(d2)


def _sc_body(d_hbm, pts_hbm, gt_hbm, t_hbm, need_hbm,
             nn_out, feat_out, gt_out,
             dbuf, chbuf, gtbuf, key_a, idx_a, key_b, idx_b,
             outv, outvi, hist, offs, tvec, nvec, sem0, sem1, sem2):
    wid = lax.axis_index("s") * 2 + lax.axis_index("c")
    row = wid
    b = row // 8

    pltpu.sync_copy(d_hbm.at[row], dbuf)
    pltpu.sync_copy(t_hbm.at[row], tvec)
    pltpu.sync_copy(need_hbm.at[row], nvec)

    iota16 = lax.iota(jnp.int32, 16)
    czero, _ = plsc.scan_count(jnp.zeros((16,), jnp.int32))
    b0vec = czero - iota16

    tval_vec = tvec[...]
    need_vec = nvec[...]

    @plsc.parallel_loop(0, _N // 16, unroll=8,
                        carry=(jnp.int32(0), jnp.int32(0)))
    def _compact(i, carry):
        nsel, ntie = carry
        key_s = _sortable_key(dbuf[pl.ds(i * 16, 16)])
        isl = key_s < tval_vec
        ise = key_s == tval_vec
        ise_i = ise.astype(jnp.int32)
        tie_pfx = plsc.cumsum(ise_i) - ise_i + ntie
        sel = jnp.logical_or(isl, jnp.logical_and(ise, tie_pfx < need_vec))
        sel_i = sel.astype(jnp.int32)
        pos = plsc.cumsum(sel_i) - sel_i + nsel
        plsc.store_scatter(key_a, [pos], key_s ^ _MIN32, mask=sel)
        plsc.store_scatter(idx_a, [pos], iota16 + i * 16, mask=sel)
        return nsel + jnp.sum(sel_i), ntie + jnp.sum(ise_i)

    fbufs = [dbuf, chbuf]
    h_ch = pltpu.async_copy(pts_hbm.at[b, 0], dbuf, sem0)
    h_gt = pltpu.async_copy(gt_hbm.at[b, 0], gtbuf, sem2)

    def _zero_hist():
        @plsc.parallel_loop(0, _NBINS // 16, unroll=8)
        def _zero(j):
            hist[pl.ds(j * 16, 16)] = jnp.zeros((16,), jnp.int32)

    _zero_hist()

    def hstep(v, _):
        kb = key_a[pl.ds(v * 16, 16)]
        dg = kb & (_NBINS - 1)
        cnt, last = plsc.scan_count(dg)
        plsc.addupdate_scatter(hist, [dg], cnt - b0vec + 1, mask=last)
        return 0

    lax.fori_loop(0, _NCROP // 16, hstep, 0, unroll=4)

    bufs = [(key_a, idx_a), (key_b, idx_b)]
    for p in range(3):
        src_k, src_i = bufs[p % 2]
        dst_k, dst_i = bufs[(p + 1) % 2]
        shift = 11 * p
        nshift = 11 * (p + 1)

        def pstep(j, carry):
            h = hist[pl.ds(j * 16, 16)]
            cs = plsc.cumsum(h)
            offs[pl.ds(j * 16, 16)] = cs - h + carry
            return carry + jnp.sum(h)

        lax.fori_loop(0, _NBINS // 16, pstep, jnp.int32(0), unroll=4)
        _zero_hist()

        def mstep(v, _, src_k=src_k, src_i=src_i, dst_k=dst_k, dst_i=dst_i,
                  shift=shift, nshift=nshift, last_pass=(p == 2)):
            kb = src_k[pl.ds(v * 16, 16)]
            ix = src_i[pl.ds(v * 16, 16)]
            dg = lax.shift_right_logical(kb, shift) & (_NBINS - 1)
            base = plsc.load_gather(offs, [dg])
            cnt, last = plsc.scan_count(dg)
            rank0 = cnt - b0vec
            pos = base + rank0
            plsc.store_scatter(dst_k, [pos], kb)
            plsc.store_scatter(dst_i, [pos], ix)
            plsc.addupdate_scatter(offs, [dg], rank0 + 1, mask=last)
            if not last_pass:
                ndg = lax.shift_right_logical(kb, nshift) & (_NBINS - 1)
                ncnt, nlast = plsc.scan_count(ndg)
                plsc.addupdate_scatter(hist, [ndg], ncnt - b0vec + 1,
                                       mask=nlast)
            return 0

        lax.fori_loop(0, _NCROP // 16, mstep, 0, unroll=4)

    pltpu.sync_copy(idx_b, nn_out.at[row])

    sems = [sem0, sem1]
    for ch in range(6):
        cur = fbufs[ch % 2]
        h_ch.wait()
        if ch + 1 < 6:
            h_ch = pltpu.async_copy(pts_hbm.at[b, ch + 1],
                                    fbufs[(ch + 1) % 2], sems[(ch + 1) % 2])

        @plsc.parallel_loop(0, _NCROP // 16, unroll=8)
        def _gather(v, cur=cur):
            ix = idx_b[pl.ds(v * 16, 16)]
            outv[pl.ds(v * 16, 16)] = plsc.load_gather(cur, [ix])

        pltpu.sync_copy(outv, feat_out.at[row, ch])

    h_gt.wait()

    @plsc.parallel_loop(0, _NCROP // 16, unroll=8)
    def _gather_gt(v):
        ix = idx_b[pl.ds(v * 16, 16)]
        outvi[pl.ds(v * 16, 16)] = plsc.load_gather(gtbuf, [ix])

    pltpu.sync_copy(outvi, gt_out.at[row])


def _sc_select_gather(d2, pts2, gt2, t_b, need_b):
    mesh = plsc.VectorSubcoreMesh(core_axis_name="c", subcore_axis_name="s")
    fn = functools.partial(
        pl.kernel,
        out_type=(jax.ShapeDtypeStruct((_ROWS, _NCROP), jnp.int32),
                  jax.ShapeDtypeStruct((_ROWS, 6, _NCROP), jnp.float32),
                  jax.ShapeDtypeStruct((_ROWS, _NCROP), jnp.int32)),
        mesh=mesh,
        compiler_params=pltpu.CompilerParams(needs_layout_passes=False),
        scratch_types=[
            pltpu.VMEM((_N,), jnp.float32),
            pltpu.VMEM((_N,), jnp.float32),
            pltpu.VMEM((_N,), jnp.int32),
            pltpu.VMEM((_NCROP,), jnp.int32),
            pltpu.VMEM((_NCROP,), jnp.int32),
            pltpu.VMEM((_NCROP,), jnp.int32),
            pltpu.VMEM((_NCROP,), jnp.int32),
            pltpu.VMEM((_NCROP,), jnp.float32),
            pltpu.VMEM((_NCROP,), jnp.int32),
            pltpu.VMEM((_NBINS,), jnp.int32),
            pltpu.VMEM((_NBINS,), jnp.int32),
            pltpu.VMEM((16,), jnp.int32),
            pltpu.VMEM((16,), jnp.int32),
            pltpu.SemaphoreType.DMA,
            pltpu.SemaphoreType.DMA,
            pltpu.SemaphoreType.DMA,
        ],
    )(_sc_body)
    return fn(d2, pts2, gt2, t_b, need_b)


def _ddf_body(feat_ref, cent_ref, out_ref):
    f = feat_ref[...]
    c = cent_ref[...]
    x, y, z = f[:, 0:1, :], f[:, 1:2, :], f[:, 2:3, :]
    cx, cy, cz = c[:, :, 0:1], c[:, :, 1:2], c[:, :, 2:3]
    dot = cx * x
    dot = dot + cy * y
    dot = dot + cz * z
    psq = x * x + y * y + z * z
    csq = cx * cx + cy * cy + cz * cz
    dd = -2.0 * dot
    dd = dd + psq
    dd = dd + csq
    ddf = jnp.exp(-4.0 * jnp.sqrt(jnp.maximum(dd, 1e-12)))
    out_ref[:, 0:3, :] = f[:, 0:3, :]
    out_ref[:, 3:9, :] = f
    out_ref[:, 9:10, :] = ddf


def _assemble(feat, centers):
    rb = 8
    return pl.pallas_call(
        _ddf_body,
        grid=(_ROWS // rb,),
        in_specs=[
            pl.BlockSpec((rb, 6, _NCROP), lambda r: (r, 0, 0)),
            pl.BlockSpec((rb, 1, 3), lambda r: (r, 0, 0)),
        ],
        out_specs=pl.BlockSpec((rb, 10, _NCROP), lambda r: (r, 0, 0)),
        out_shape=jax.ShapeDtypeStruct((_ROWS, 10, _NCROP), jnp.float32),
        compiler_params=pltpu.CompilerParams(
            dimension_semantics=("parallel",)),
    )(feat, centers)


def kernel(points, gt_seg, center_points):
    B, C, N = points.shape
    K = center_points.shape[1]

    xyz = jnp.transpose(points[:, :3, :], (0, 2, 1))
    dd = -2.0 * jnp.einsum('bnc,bmc->bnm', center_points, xyz)
    dd = dd + jnp.sum(center_points ** 2, axis=-1)[:, :, None]
    dd = dd + jnp.sum(xyz ** 2, axis=-1)[:, None, :]

    d2 = dd.reshape(_ROWS, _N)
    t_b, need_b = _thresholds(d2)

    nn2, feat, gtg = _sc_select_gather(d2, points, gt_seg, t_b, need_b)

    out = _assemble(feat, center_points.reshape(_ROWS, 1, 3))

    cropped_feature_ls = out
    cropped_gt = gtg.reshape(_ROWS, 1, _NCROP)
    nn_idx = nn2.reshape(B, K, _NCROP)
    return (cropped_feature_ls, cropped_gt, nn_idx)


# speedup vs baseline: 1.0225x; 1.0225x over previous
import functools

import jax
import jax.numpy as jnp
from jax import lax
from jax.experimental import pallas as pl
from jax.experimental.pallas import tpu as pltpu
from jax.experimental.pallas import tpu_sc as plsc

_N = 24000
_NCROP = 3072
_ROWS = 32
_NBINS = 2048
_MIN32 = -2147483648
_FLIP = 0x7FFFFFFF


def _sortable_key(dv):
    kv = lax.bitcast_convert_type(dv, jnp.int32)
    return jnp.where(kv < 0, kv ^ _FLIP, kv)


def _thresh_body(d_ref, t_ref, need_ref):
    kb = _sortable_key(d_ref[...]) ^ _MIN32
    nr = kb.shape[0]

    def step(i, carry):
        tval, r, cl, dm = carry
        bit = 31 - i
        bmask = jnp.int32(1) << bit
        matched = (kb & dm) == tval
        bitzero = (kb & bmask) == 0
        c0 = jnp.sum((matched & bitzero).astype(jnp.int32), axis=1,
                     keepdims=True)
        choose0 = r < c0
        tval = jnp.where(choose0, tval, tval | bmask)
        r = jnp.where(choose0, r, r - c0)
        cl = jnp.where(choose0, cl, cl + c0)
        return tval, r, cl, dm | bmask

    z = jnp.zeros((nr, 1), jnp.int32)
    tval, _, cl, _ = lax.fori_loop(
        0, 32, step, (z, z + jnp.int32(_NCROP - 1), z, jnp.int32(0)))
    t_s = tval ^ _MIN32
    t_ref[...] = jnp.broadcast_to(t_s, (nr, 16))
    need_ref[...] = jnp.broadcast_to(jnp.int32(_NCROP) - cl, (nr, 16))


def _thresholds(d2):
    half = _ROWS // 2
    return pl.pallas_call(
        _thresh_body,
        grid=(2,),
        in_specs=[pl.BlockSpec((half, _N), lambda i: (i, 0))],
        out_specs=(pl.BlockSpec((half, 16), lambda i: (i, 0)),
                   pl.BlockSpec((half, 16), lambda i: (i, 0))),
        out_shape=(jax.ShapeDtypeStruct((_ROWS, 16), jnp.int32),
                   jax.ShapeDtypeStruct((_ROWS, 16), jnp.int32)),
        compiler_params=pltpu.CompilerParams(
            dimension_semantics=("parallel",)),
    )(d2)


def _sc_body(d_hbm, pts_hbm, gt_hbm, t_hbm, need_hbm,
             nn_out, feat_out, gt_out,
             dbuf, chbuf, gtbuf, key_a, idx_a, key_b, idx_b,
             outv, outvi, hist, offs, tvec, nvec, sem0, sem1, sem2):
    wid = lax.axis_index("s") * 2 + lax.axis_index("c")
    row = wid
    b = row // 8

    pltpu.sync_copy(d_hbm.at[row], dbuf)
    pltpu.sync_copy(t_hbm.at[row], tvec)
    pltpu.sync_copy(need_hbm.at[row], nvec)

    iota16 = lax.iota(jnp.int32, 16)
    czero, _ = plsc.scan_count(jnp.zeros((16,), jnp.int32))
    b0vec = czero - iota16

    tval_vec = tvec[...]
    need_vec = nvec[...]

    @plsc.parallel_loop(0, _N // 16, unroll=8,
                        carry=(jnp.int32(0), jnp.int32(0)))
    def _compact(i, carry):
        nsel, ntie = carry
        key_s = _sortable_key(dbuf[pl.ds(i * 16, 16)])
        isl = key_s < tval_vec
        ise = key_s == tval_vec
        ise_i = ise.astype(jnp.int32)
        tie_pfx = plsc.cumsum(ise_i) - ise_i + ntie
        sel = jnp.logical_or(isl, jnp.logical_and(ise, tie_pfx < need_vec))
        sel_i = sel.astype(jnp.int32)
        pos = plsc.cumsum(sel_i) - sel_i + nsel
        plsc.store_scatter(key_a, [pos], key_s ^ _MIN32, mask=sel)
        plsc.store_scatter(idx_a, [pos], iota16 + i * 16, mask=sel)
        return nsel + jnp.sum(sel_i), ntie + jnp.sum(ise_i)

    fbufs = [dbuf, chbuf]
    h_ch = pltpu.async_copy(pts_hbm.at[b, 0], dbuf, sem0)
    h_gt = pltpu.async_copy(gt_hbm.at[b, 0], gtbuf, sem2)

    def _zero_hist():
        @plsc.parallel_loop(0, _NBINS // 16, unroll=8)
        def _zero(j):
            hist[pl.ds(j * 16, 16)] = jnp.zeros((16,), jnp.int32)

    _zero_hist()

    def hstep(v, _):
        kb = key_a[pl.ds(v * 16, 16)]
        dg = kb & (_NBINS - 1)
        cnt, last = plsc.scan_count(dg)
        plsc.addupdate_scatter(hist, [dg], cnt - b0vec + 1, mask=last)
        return 0

    lax.fori_loop(0, _NCROP // 16, hstep, 0)

    bufs = [(key_a, idx_a), (key_b, idx_b)]
    for p in range(3):
        src_k, src_i = bufs[p % 2]
        dst_k, dst_i = bufs[(p + 1) % 2]
        shift = 11 * p
        nshift = 11 * (p + 1)

        def pstep(j, carry):
            h = hist[pl.ds(j * 16, 16)]
            cs = plsc.cumsum(h)
            offs[pl.ds(j * 16, 16)] = cs - h + carry
            return carry + jnp.sum(h)

        lax.fori_loop(0, _NBINS // 16, pstep, jnp.int32(0))
        _zero_hist()

        def mstep(v, _, src_k=src_k, src_i=src_i, dst_k=dst_k, dst_i=dst_i,
                  shift=shift, nshift=nshift, last_pass=(p == 2)):
            kb = src_k[pl.ds(v * 16, 16)]
            ix = src_i[pl.ds(v * 16, 16)]
            dg = lax.shift_right_logical(kb, shift) & (_NBINS - 1)
            base = plsc.load_gather(offs, [dg])
            cnt, last = plsc.scan_count(dg)
            rank0 = cnt - b0vec
            pos = base + rank0
            plsc.store_scatter(dst_k, [pos], kb)
            plsc.store_scatter(dst_i, [pos], ix)
            plsc.addupdate_scatter(offs, [dg], rank0 + 1, mask=last)
            if not last_pass:
                ndg = lax.shift_right_logical(kb, nshift) & (_NBINS - 1)
                ncnt, nlast = plsc.scan_count(ndg)
                plsc.addupdate_scatter(hist, [ndg], ncnt - b0vec + 1,
                                       mask=nlast)
            return 0

        lax.fori_loop(0, _NCROP // 16, mstep, 0)

    pltpu.sync_copy(idx_b, nn_out.at[row])

    sems = [sem0, sem1]
    for ch in range(6):
        cur = fbufs[ch % 2]
        h_ch.wait()
        if ch + 1 < 6:
            h_ch = pltpu.async_copy(pts_hbm.at[b, ch + 1],
                                    fbufs[(ch + 1) % 2], sems[(ch + 1) % 2])

        @plsc.parallel_loop(0, _NCROP // 16, unroll=8)
        def _gather(v, cur=cur):
            ix = idx_b[pl.ds(v * 16, 16)]
            outv[pl.ds(v * 16, 16)] = plsc.load_gather(cur, [ix])

        pltpu.sync_copy(outv, feat_out.at[row, ch])

    h_gt.wait()

    @plsc.parallel_loop(0, _NCROP // 16, unroll=8)
    def _gather_gt(v):
        ix = idx_b[pl.ds(v * 16, 16)]
        outvi[pl.ds(v * 16, 16)] = plsc.load_gather(gtbuf, [ix])

    pltpu.sync_copy(outvi, gt_out.at[row])


def _sc_select_gather(d2, pts2, gt2, t_b, need_b):
    mesh = plsc.VectorSubcoreMesh(core_axis_name="c", subcore_axis_name="s")
    fn = functools.partial(
        pl.kernel,
        out_type=(jax.ShapeDtypeStruct((_ROWS, _NCROP), jnp.int32),
                  jax.ShapeDtypeStruct((_ROWS, 6, _NCROP), jnp.float32),
                  jax.ShapeDtypeStruct((_ROWS, _NCROP), jnp.int32)),
        mesh=mesh,
        compiler_params=pltpu.CompilerParams(needs_layout_passes=False),
        scratch_types=[
            pltpu.VMEM((_N,), jnp.float32),
            pltpu.VMEM((_N,), jnp.float32),
            pltpu.VMEM((_N,), jnp.int32),
            pltpu.VMEM((_NCROP,), jnp.int32),
            pltpu.VMEM((_NCROP,), jnp.int32),
            pltpu.VMEM((_NCROP,), jnp.int32),
            pltpu.VMEM((_NCROP,), jnp.int32),
            pltpu.VMEM((_NCROP,), jnp.float32),
            pltpu.VMEM((_NCROP,), jnp.int32),
            pltpu.VMEM((_NBINS,), jnp.int32),
            pltpu.VMEM((_NBINS,), jnp.int32),
            pltpu.VMEM((16,), jnp.int32),
            pltpu.VMEM((16,), jnp.int32),
            pltpu.SemaphoreType.DMA,
            pltpu.SemaphoreType.DMA,
            pltpu.SemaphoreType.DMA,
        ],
    )(_sc_body)
    return fn(d2, pts2, gt2, t_b, need_b)


def _ddf_body(feat_ref, cent_ref, out_ref):
    f = feat_ref[...]
    c = cent_ref[...]
    x, y, z = f[:, 0:1, :], f[:, 1:2, :], f[:, 2:3, :]
    cx, cy, cz = c[:, :, 0:1], c[:, :, 1:2], c[:, :, 2:3]
    dot = cx * x
    dot = dot + cy * y
    dot = dot + cz * z
    psq = x * x + y * y + z * z
    csq = cx * cx + cy * cy + cz * cz
    dd = -2.0 * dot
    dd = dd + psq
    dd = dd + csq
    ddf = jnp.exp(-4.0 * jnp.sqrt(jnp.maximum(dd, 1e-12)))
    out_ref[:, 0:3, :] = f[:, 0:3, :]
    out_ref[:, 3:9, :] = f
    out_ref[:, 9:10, :] = ddf


def _assemble(feat, centers):
    rb = 8
    return pl.pallas_call(
        _ddf_body,
        grid=(_ROWS // rb,),
        in_specs=[
            pl.BlockSpec((rb, 6, _NCROP), lambda r: (r, 0, 0)),
            pl.BlockSpec((rb, 1, 3), lambda r: (r, 0, 0)),
        ],
        out_specs=pl.BlockSpec((rb, 10, _NCROP), lambda r: (r, 0, 0)),
        out_shape=jax.ShapeDtypeStruct((_ROWS, 10, _NCROP), jnp.float32),
        compiler_params=pltpu.CompilerParams(
            dimension_semantics=("parallel",)),
    )(feat, centers)


def kernel(points, gt_seg, center_points):
    B, C, N = points.shape
    K = center_points.shape[1]

    xyz = jnp.transpose(points[:, :3, :], (0, 2, 1))
    dd = -2.0 * jnp.einsum('bnc,bmc->bnm', center_points, xyz)
    dd = dd + jnp.sum(center_points ** 2, axis=-1)[:, :, None]
    dd = dd + jnp.sum(xyz ** 2, axis=-1)[:, None, :]

    d2 = dd.reshape(_ROWS, _N)
    t_b, need_b = _thresholds(d2)

    nn2, feat, gtg = _sc_select_gather(d2, points, gt_seg, t_b, need_b)

    out = _assemble(feat, center_points.reshape(_ROWS, 1, 3))

    cropped_feature_ls = out
    cropped_gt = gtg.reshape(_ROWS, 1, _NCROP)
    nn_idx = nn2.reshape(B, K, _NCROP)
    return (cropped_feature_ls, cropped_gt, nn_idx)
